# Initial kernel scaffold; baseline (speedup 1.0000x reference)
import jax
import jax.numpy as jnp
from jax.experimental import pallas as pl


def kernel(first_char, digits, mods, vals, kappa, first_level_w, integer_w):
    raise NotImplementedError("write your pallas kernel here")


# speedup vs baseline: 4.3657x; 4.3657x over previous
import functools

import jax
import jax.numpy as jnp
from jax import lax
from jax.experimental import pallas as pl
from jax.experimental.pallas import tpu as pltpu
from jax.experimental.pallas import tpu_sc as plsc

N = 819200
L = 6
D = 64
ROWS = 2 * 26 + 1000
PAIR_ROWS = 300

NC = 2
NS = 16
NW = NC * NS
PER_W = N // NW
CP = 256
NCHUNK = PER_W // CP
G = CP // 16


def _sc_body(fc_h, dg_h, md_h, vl_h, tab_h, pair_h, out_h,
             tabv, pairv, fcv, dgv, mdv, vlv, outv):
    c = lax.axis_index("c")
    s = lax.axis_index("s")
    wid = s * NC + c
    base0 = wid * PER_W

    pltpu.sync_copy(tab_h, tabv)
    pltpu.sync_copy(pair_h, pairv)

    iot = lax.iota(jnp.int32, 16)

    def chunk_body(j, carry):
        gb = base0 + j * CP
        pltpu.sync_copy(fc_h.at[pl.ds(gb, CP)], fcv)
        pltpu.sync_copy(dg_h.at[pl.ds(gb * L, CP * L)], dgv)
        pltpu.sync_copy(md_h.at[pl.ds(gb, CP)], mdv)
        pltpu.sync_copy(vl_h.at[pl.ds(gb, CP)], vlv)

        def group_body(g, carry2):
            t0 = g * 16
            tok = t0 + iot
            fc16 = fcv[pl.ds(t0, 16)]
            md16 = mdv[pl.ds(t0, 16)]
            vl16 = vlv[pl.ds(t0, 16)]
            tok6 = tok * L
            d0 = plsc.load_gather(dgv, [tok6])
            d1 = plsc.load_gather(dgv, [tok6 + 1])
            d2 = plsc.load_gather(dgv, [tok6 + 2])
            d3 = plsc.load_gather(dgv, [tok6 + 3])
            d4 = plsc.load_gather(dgv, [tok6 + 4])
            d5 = plsc.load_gather(dgv, [tok6 + 5])
            rb = fc16 * D
            b01 = (d0 * 10 + d1) * D
            b23 = (d2 * 10 + d3 + 100) * D
            b45 = (d4 * 10 + d5 + 200) * D
            ob = tok * D
            is_lab = md16 == 2
            scale = jnp.where(is_lab, vl16, jnp.ones_like(vl16))
            dmask = jnp.where(is_lab, jnp.zeros_like(vl16), jnp.ones_like(vl16))
            for d in range(D):
                b = plsc.load_gather(tabv, [rb + d])
                p1 = plsc.load_gather(pairv, [b01 + d])
                p2 = plsc.load_gather(pairv, [b23 + d])
                p3 = plsc.load_gather(pairv, [b45 + d])
                o = b * scale + dmask * ((p1 + p2) + p3)
                plsc.store_scatter(outv, [ob + d], o)
            return carry2

        lax.fori_loop(0, G, group_body, 0)
        pltpu.sync_copy(outv, out_h.at[pl.ds(gb * D, CP * D)])
        return carry

    lax.fori_loop(0, NCHUNK, chunk_body, 0)


@functools.partial(
    pl.kernel,
    out_type=jax.ShapeDtypeStruct((N * D,), jnp.float32),
    mesh=plsc.VectorSubcoreMesh(core_axis_name="c", subcore_axis_name="s"),
    compiler_params=pltpu.CompilerParams(needs_layout_passes=False),
    scratch_types=[
        pltpu.VMEM((ROWS * D,), jnp.float32),
        pltpu.VMEM((PAIR_ROWS * D,), jnp.float32),
        pltpu.VMEM((CP,), jnp.int32),
        pltpu.VMEM((CP * L,), jnp.int32),
        pltpu.VMEM((CP,), jnp.int32),
        pltpu.VMEM((CP,), jnp.float32),
        pltpu.VMEM((CP * D,), jnp.float32),
    ],
)
def _sc_kernel(fc_h, dg_h, md_h, vl_h, tab_h, pair_h, out_h,
               tabv, pairv, fcv, dgv, mdv, vlv, outv):
    _sc_body(fc_h, dg_h, md_h, vl_h, tab_h, pair_h, out_h,
             tabv, pairv, fcv, dgv, mdv, vlv, outv)


def kernel(first_char, digits, mods, vals, kappa, first_level_w, integer_w):
    pos_w = 1.0 / (jnp.arange(L, dtype=jnp.float32) + 2.0) ** jnp.asarray(
        kappa, jnp.float32)
    pair = jnp.concatenate(
        [
            (pos_w[2 * k] * integer_w[:, None, :]
             + pos_w[2 * k + 1] * integer_w[None, :, :]).reshape(100, D)
            for k in range(3)
        ],
        axis=0,
    )
    fc = first_char.astype(jnp.int32)
    dg = digits.astype(jnp.int32).reshape(-1)
    md = mods.astype(jnp.int32)
    vl = vals.astype(jnp.float32)
    out = _sc_kernel(fc, dg, md, vl,
                     first_level_w.reshape(-1), pair.reshape(-1))
    return out.reshape(N, D)


# speedup vs baseline: 6.1996x; 1.4201x over previous
import functools

import jax
import jax.numpy as jnp
from jax import lax
from jax.experimental import pallas as pl
from jax.experimental.pallas import tpu as pltpu
from jax.experimental.pallas import tpu_sc as plsc

N = 819200
L = 6
D = 64
ROWS = 2 * 26 + 1000
PAIR_ROWS = 300

NC = 2
NS = 16
NW = NC * NS
PER_W = N // NW
CP = 256
NCHUNK = PER_W // CP
NP = NCHUNK // 2
G = CP // 16

TAB_LEN = ROWS * D
TAB_SL = TAB_LEN - D + 1
PAIR_SL = 100 * D - D + 1
DG_SL = CP * L - (L - 1)
OUT_SL = CP * D - D + 1


def _sc_body(fc_h, dg_h, md_h, vl_h, tab_h, pair_h, out_h,
             tabv, pairv, fcv, dgv, mdv, vlv, outv,
             ins0, ins1, outs0, outs1):
    c = lax.axis_index("c")
    s = lax.axis_index("s")
    wid = s * NC + c
    base0 = wid * PER_W

    pltpu.sync_copy(tab_h, tabv)
    pltpu.sync_copy(pair_h, pairv)

    iot = lax.iota(jnp.int32, 16)
    insems = (ins0, ins1)
    outsems = (outs0, outs1)

    def in_copies(ch, b):
        gb = base0 + ch * CP
        return (
            (fc_h.at[pl.ds(gb, CP)], fcv.at[pl.ds(b * CP, CP)]),
            (dg_h.at[pl.ds(gb * L, CP * L)], dgv.at[pl.ds(b * CP * L, CP * L)]),
            (md_h.at[pl.ds(gb, CP)], mdv.at[pl.ds(b * CP, CP)]),
            (vl_h.at[pl.ds(gb, CP)], vlv.at[pl.ds(b * CP, CP)]),
        )

    def start_in(ch, b):
        for src, dst in in_copies(ch, b):
            pltpu.async_copy(src, dst, insems[b])

    def drain_in(ch, b):
        for src, dst in in_copies(ch, b):
            pltpu.make_async_copy(src, dst, insems[b]).wait()

    def out_copy(ch, b):
        gb = base0 + ch * CP
        return (outv.at[pl.ds(b * CP * D, CP * D)],
                out_h.at[pl.ds(gb * D, CP * D)])

    def compute(b):
        ob_off = b * CP * D

        @plsc.parallel_loop(0, G, unroll=1)
        def group_body(g):
            t0 = b * CP + g * 16
            local = g * 16 + iot
            fc16 = fcv[pl.ds(t0, 16)]
            md16 = mdv[pl.ds(t0, 16)]
            vl16 = vlv[pl.ds(t0, 16)]
            t6 = local * L
            dgb = b * CP * L
            dgslice = dgv.at[pl.ds(dgb, CP * L)]
            d0 = plsc.load_gather(dgslice, [t6])
            d1 = plsc.load_gather(dgslice, [t6 + 1])
            d2 = plsc.load_gather(dgslice, [t6 + 2])
            d3 = plsc.load_gather(dgslice, [t6 + 3])
            d4 = plsc.load_gather(dgslice, [t6 + 4])
            d5 = plsc.load_gather(dgslice, [t6 + 5])
            rb = fc16 * D
            b01 = (d0 * 10 + d1) * D
            b23 = (d2 * 10 + d3) * D
            b45 = (d4 * 10 + d5) * D
            ob = local * D
            is_lab = md16 == 2
            scale = jnp.where(is_lab, vl16, jnp.ones_like(vl16))
            dmask = jnp.where(is_lab, jnp.zeros_like(vl16), jnp.ones_like(vl16))
            for r in range(8):
                rbr = rb + r if r else rb
                b01r = b01 + r if r else b01
                b23r = b23 + r if r else b23
                b45r = b45 + r if r else b45
                obr = ob + r if r else ob
                for q in range(8):
                    o8 = 8 * q
                    bcol = plsc.load_gather(
                        tabv.at[pl.ds(o8, TAB_LEN - o8)], [rbr])
                    p1 = plsc.load_gather(
                        pairv.at[pl.ds(o8, 6400 - o8)], [b01r])
                    p2 = plsc.load_gather(
                        pairv.at[pl.ds(6400 + o8, 6400 - o8)], [b23r])
                    p3 = plsc.load_gather(
                        pairv.at[pl.ds(12800 + o8, 6400 - o8)], [b45r])
                    o = bcol * scale + dmask * ((p1 + p2) + p3)
                    plsc.store_scatter(
                        outv.at[pl.ds(ob_off + o8, CP * D - o8)], [obr], o)

    start_in(0, 0)
    start_in(1, 1)

    def pair_body(jp, carry):
        for bbuf in (0, 1):
            ch = 2 * jp + bbuf
            drain_in(ch, bbuf)

            @pl.when(jp >= 1)
            def _wait_out():
                src, dst = out_copy(ch - 2, bbuf)
                pltpu.make_async_copy(src, dst, outsems[bbuf]).wait()

            compute(bbuf)
            src, dst = out_copy(ch, bbuf)
            pltpu.async_copy(src, dst, outsems[bbuf])

            @pl.when(jp < NP - 1)
            def _prefetch():
                start_in(ch + 2, bbuf)
        return carry

    lax.fori_loop(0, NP, pair_body, 0)
    for bbuf in (0, 1):
        src, dst = out_copy(NCHUNK - 2 + bbuf, bbuf)
        pltpu.make_async_copy(src, dst, outsems[bbuf]).wait()


@functools.partial(
    pl.kernel,
    out_type=jax.ShapeDtypeStruct((N * D,), jnp.float32),
    mesh=plsc.VectorSubcoreMesh(core_axis_name="c", subcore_axis_name="s"),
    compiler_params=pltpu.CompilerParams(needs_layout_passes=False),
    scratch_types=[
        pltpu.VMEM((TAB_LEN,), jnp.float32),
        pltpu.VMEM((PAIR_ROWS * D,), jnp.float32),
        pltpu.VMEM((2 * CP,), jnp.int32),
        pltpu.VMEM((2 * CP * L,), jnp.int32),
        pltpu.VMEM((2 * CP,), jnp.int32),
        pltpu.VMEM((2 * CP,), jnp.float32),
        pltpu.VMEM((2 * CP * D,), jnp.float32),
        pltpu.SemaphoreType.DMA,
        pltpu.SemaphoreType.DMA,
        pltpu.SemaphoreType.DMA,
        pltpu.SemaphoreType.DMA,
    ],
)
def _sc_kernel(fc_h, dg_h, md_h, vl_h, tab_h, pair_h, out_h,
               tabv, pairv, fcv, dgv, mdv, vlv, outv,
               ins0, ins1, outs0, outs1):
    _sc_body(fc_h, dg_h, md_h, vl_h, tab_h, pair_h, out_h,
             tabv, pairv, fcv, dgv, mdv, vlv, outv,
             ins0, ins1, outs0, outs1)


def kernel(first_char, digits, mods, vals, kappa, first_level_w, integer_w):
    pos_w = 1.0 / (jnp.arange(L, dtype=jnp.float32) + 2.0) ** jnp.asarray(
        kappa, jnp.float32)
    pair = jnp.concatenate(
        [
            (pos_w[2 * k] * integer_w[:, None, :]
             + pos_w[2 * k + 1] * integer_w[None, :, :]).reshape(100, D)
            for k in range(3)
        ],
        axis=0,
    )
    fc = first_char.astype(jnp.int32)
    dg = digits.astype(jnp.int32).reshape(-1)
    md = mods.astype(jnp.int32)
    vl = vals.astype(jnp.float32)
    out = _sc_kernel(fc, dg, md, vl,
                     first_level_w.reshape(-1), pair.reshape(-1))
    return out.reshape(N, D)


# speedup vs baseline: 14.0067x; 2.2593x over previous
import functools

import jax
import jax.numpy as jnp
from jax import lax
from jax.experimental import pallas as pl
from jax.experimental.pallas import tpu as pltpu
from jax.experimental.pallas import tpu_sc as plsc

N = 819200
L = 6
D = 64
ST = 65
ROWS = 2 * 26 + 1000

NC = 2
NS = 16
NW = NC * NS
PER_W = N // NW
CP = 256
NCHUNK = PER_W // CP
NP = NCHUNK // 2
G = CP // 16

TAB_LEN = ROWS * ST
PAIR_LEN = 100 * ST


def _sc_body(fc_h, dg_h, md_h, vl_h, tab_h, p0_h, p1_h, p2_h, out_h,
             tabv, p0v, p1v, p2v, fcv, dgv, mdv, vlv, outv,
             ins0, ins1, outs0, outs1):
    c = lax.axis_index("c")
    s = lax.axis_index("s")
    wid = s * NC + c
    base0 = wid * PER_W

    pltpu.sync_copy(tab_h, tabv)
    pltpu.sync_copy(p0_h, p0v)
    pltpu.sync_copy(p1_h, p1v)
    pltpu.sync_copy(p2_h, p2v)

    iot = lax.iota(jnp.int32, 16)
    insems = (ins0, ins1)
    outsems = (outs0, outs1)

    def in_copies(ch, b):
        gb = base0 + ch * CP
        return (
            (fc_h.at[pl.ds(gb, CP)], fcv.at[pl.ds(b * CP, CP)]),
            (dg_h.at[pl.ds(gb * L, CP * L)], dgv.at[pl.ds(b * CP * L, CP * L)]),
            (md_h.at[pl.ds(gb, CP)], mdv.at[pl.ds(b * CP, CP)]),
            (vl_h.at[pl.ds(gb, CP)], vlv.at[pl.ds(b * CP, CP)]),
        )

    def start_in(ch, b):
        for src, dst in in_copies(ch, b):
            pltpu.async_copy(src, dst, insems[b])

    def drain_in(ch, b):
        for src, dst in in_copies(ch, b):
            pltpu.make_async_copy(src, dst, insems[b]).wait()

    def out_copy(ch, b):
        gb = base0 + ch * CP
        return (outv.at[pl.ds(b * CP * D, CP * D)],
                out_h.at[pl.ds(gb * D, CP * D)])

    def compute(b):
        ob_off = b * CP * D

        @plsc.parallel_loop(0, G, unroll=1)
        def group_body(g):
            t0 = b * CP + g * 16
            local = g * 16 + iot
            fc16 = fcv[pl.ds(t0, 16)]
            md16 = mdv[pl.ds(t0, 16)]
            vl16 = vlv[pl.ds(t0, 16)]
            t6 = local * L
            dgslice = dgv.at[pl.ds(b * CP * L, CP * L)]
            d0 = plsc.load_gather(dgslice, [t6])
            d1 = plsc.load_gather(dgslice, [t6 + 1])
            d2 = plsc.load_gather(dgslice, [t6 + 2])
            d3 = plsc.load_gather(dgslice, [t6 + 3])
            d4 = plsc.load_gather(dgslice, [t6 + 4])
            d5 = plsc.load_gather(dgslice, [t6 + 5])
            rb = fc16 * ST
            b01 = (d0 * 10 + d1) * ST
            b23 = (d2 * 10 + d3) * ST
            b45 = (d4 * 10 + d5) * ST
            ob = local * D
            is_lab = md16 == 2
            scale = jnp.where(is_lab, vl16, jnp.ones_like(vl16))
            dmask = jnp.where(is_lab, jnp.zeros_like(vl16), jnp.ones_like(vl16))
            for r in range(8):
                rbr = rb + r if r else rb
                b01r = b01 + r if r else b01
                b23r = b23 + r if r else b23
                b45r = b45 + r if r else b45
                obr = ob + r if r else ob
                for q in range(8):
                    o8 = 8 * q
                    bcol = plsc.load_gather(
                        tabv.at[pl.ds(o8, TAB_LEN - o8)], [rbr])
                    p1 = plsc.load_gather(
                        p0v.at[pl.ds(o8, PAIR_LEN - o8)], [b01r])
                    p2 = plsc.load_gather(
                        p1v.at[pl.ds(o8, PAIR_LEN - o8)], [b23r])
                    p3 = plsc.load_gather(
                        p2v.at[pl.ds(o8, PAIR_LEN - o8)], [b45r])
                    o = bcol * scale + dmask * ((p1 + p2) + p3)
                    plsc.store_scatter(
                        outv.at[pl.ds(ob_off + o8, CP * D - o8)], [obr], o)

    start_in(0, 0)
    start_in(1, 1)

    def pair_body(jp, carry):
        for bbuf in (0, 1):
            ch = 2 * jp + bbuf
            drain_in(ch, bbuf)

            @pl.when(jp >= 1)
            def _wait_out():
                src, dst = out_copy(ch - 2, bbuf)
                pltpu.make_async_copy(src, dst, outsems[bbuf]).wait()

            compute(bbuf)
            src, dst = out_copy(ch, bbuf)
            pltpu.async_copy(src, dst, outsems[bbuf])

            @pl.when(jp < NP - 1)
            def _prefetch():
                start_in(ch + 2, bbuf)
        return carry

    lax.fori_loop(0, NP, pair_body, 0)
    for bbuf in (0, 1):
        src, dst = out_copy(NCHUNK - 2 + bbuf, bbuf)
        pltpu.make_async_copy(src, dst, outsems[bbuf]).wait()


@functools.partial(
    pl.kernel,
    out_type=jax.ShapeDtypeStruct((N * D,), jnp.float32),
    mesh=plsc.VectorSubcoreMesh(core_axis_name="c", subcore_axis_name="s"),
    compiler_params=pltpu.CompilerParams(needs_layout_passes=False),
    scratch_types=[
        pltpu.VMEM((TAB_LEN,), jnp.float32),
        pltpu.VMEM((PAIR_LEN,), jnp.float32),
        pltpu.VMEM((PAIR_LEN,), jnp.float32),
        pltpu.VMEM((PAIR_LEN,), jnp.float32),
        pltpu.VMEM((2 * CP,), jnp.int32),
        pltpu.VMEM((2 * CP * L,), jnp.int32),
        pltpu.VMEM((2 * CP,), jnp.int32),
        pltpu.VMEM((2 * CP,), jnp.float32),
        pltpu.VMEM((2 * CP * D,), jnp.float32),
        pltpu.SemaphoreType.DMA,
        pltpu.SemaphoreType.DMA,
        pltpu.SemaphoreType.DMA,
        pltpu.SemaphoreType.DMA,
    ],
)
def _sc_kernel(fc_h, dg_h, md_h, vl_h, tab_h, p0_h, p1_h, p2_h, out_h,
               tabv, p0v, p1v, p2v, fcv, dgv, mdv, vlv, outv,
               ins0, ins1, outs0, outs1):
    _sc_body(fc_h, dg_h, md_h, vl_h, tab_h, p0_h, p1_h, p2_h, out_h,
             tabv, p0v, p1v, p2v, fcv, dgv, mdv, vlv, outv,
             ins0, ins1, outs0, outs1)


def kernel(first_char, digits, mods, vals, kappa, first_level_w, integer_w):
    pos_w = 1.0 / (jnp.arange(L, dtype=jnp.float32) + 2.0) ** jnp.asarray(
        kappa, jnp.float32)
    pads = []
    for k in range(3):
        pk = (pos_w[2 * k] * integer_w[:, None, :]
              + pos_w[2 * k + 1] * integer_w[None, :, :]).reshape(100, D)
        pads.append(jnp.pad(pk, ((0, 0), (0, ST - D))).reshape(-1))
    tab = jnp.pad(first_level_w, ((0, 0), (0, ST - D))).reshape(-1)
    fc = first_char.astype(jnp.int32)
    dg = digits.astype(jnp.int32).reshape(-1)
    md = mods.astype(jnp.int32)
    vl = vals.astype(jnp.float32)
    out = _sc_kernel(fc, dg, md, vl, tab, pads[0], pads[1], pads[2])
    return out.reshape(N, D)
